# baseline (speedup 1.0000x reference)
import jax
import jax.numpy as jnp
from jax import lax
from jax.experimental import pallas as pl
from jax.experimental.pallas import tpu as pltpu
from jax.experimental.pallas import tpu_sc as plsc

B, L, DIM = 4096, 50, 64
NL = B * L
NC, NS = 2, 16
NW = NC * NS
PER_W = NL // NW
TCH = 128
NCH = PER_W // TCH
NBUF = 5
FROWS = 100001


def _sc_gather_body(fidx_hbm, cidx_hbm, ftab_hbm, ctab_hbm, x_hbm,
                    ftmp, ctmp, fbuf, cbuf, fsems, csems):
    wid = lax.axis_index("s") * NC + lax.axis_index("c")
    tbase = wid * PER_W
    pltpu.sync_copy(fidx_hbm.at[pl.ds(tbase, PER_W)], ftmp)
    pltpu.sync_copy(cidx_hbm.at[pl.ds(tbase, PER_W)], ctmp)

    def gather(j, slot):
        jc = lax.min(j, NCH - 1)
        pltpu.async_copy(ftab_hbm.at[ftmp.at[pl.ds(jc * TCH, TCH)]],
                         fbuf.at[slot], fsems.at[slot])
        pltpu.async_copy(ctab_hbm.at[ctmp.at[pl.ds(jc * TCH, TCH)]],
                         cbuf.at[slot], csems.at[slot])

    def wait_write(j, slot):
        pltpu.make_async_copy(ftab_hbm.at[ftmp.at[pl.ds(0, TCH)]],
                              fbuf.at[slot], fsems.at[slot]).wait()
        pltpu.make_async_copy(ctab_hbm.at[ctmp.at[pl.ds(0, TCH)]],
                              cbuf.at[slot], csems.at[slot]).wait()
        row0 = tbase + j * TCH
        pltpu.sync_copy(fbuf.at[slot],
                        x_hbm.at[pl.ds(row0, TCH), pl.ds(0, DIM)])
        pltpu.sync_copy(cbuf.at[slot],
                        x_hbm.at[pl.ds(row0, TCH), pl.ds(DIM, DIM)])

    for s in range(NBUF):
        gather(s, s)

    def group(jj, carry):
        j0 = jj * NBUF
        for s in range(NBUF):
            wait_write(j0 + s, s)
            gather(j0 + s + NBUF, s)
        return carry

    lax.fori_loop(0, NCH // NBUF, group, 0)
    for slot in range(NBUF):
        pltpu.make_async_copy(ftab_hbm.at[ftmp.at[pl.ds(0, TCH)]],
                              fbuf.at[slot], fsems.at[slot]).wait()
        pltpu.make_async_copy(ctab_hbm.at[ctmp.at[pl.ds(0, TCH)]],
                              cbuf.at[slot], csems.at[slot]).wait()


def _sc_gather(fidx, cidx, ftab, ctab):
    return pl.kernel(
        _sc_gather_body,
        out_type=jax.ShapeDtypeStruct((NL, 2 * DIM), jnp.bfloat16),
        mesh=plsc.VectorSubcoreMesh(core_axis_name="c", subcore_axis_name="s",
                                    num_cores=NC, num_subcores=NS),
        scratch_types=[
            pltpu.VMEM((PER_W,), jnp.int32),
            pltpu.VMEM((PER_W,), jnp.int32),
            pltpu.VMEM((NBUF, TCH, DIM), jnp.bfloat16),
            pltpu.VMEM((NBUF, TCH, DIM), jnp.bfloat16),
            pltpu.SemaphoreType.DMA((NBUF,)),
            pltpu.SemaphoreType.DMA((NBUF,)),
        ],
        compiler_params=pltpu.CompilerParams(use_tc_tiling_on_sc=False),
    )(fidx, cidx, ftab, ctab)


def _tc_mlp_body(x_ref, w1_ref, b1_ref, w2_ref, b2_ref, o_ref):
    x = x_ref[...]
    h = jnp.dot(x, w1_ref[...], preferred_element_type=jnp.float32)
    h = jnp.maximum(h + b1_ref[...], 0.0)
    y = (jnp.dot(h, w2_ref[...], preferred_element_type=jnp.float32)
         + b2_ref[...])
    o_ref[0] = jnp.transpose(y)


def _tc_mlp(x2, W1, b1, W2, b2, *, interpret=False):
    return pl.pallas_call(
        _tc_mlp_body,
        grid=(L,),
        in_specs=[
            pl.BlockSpec((B, 2 * DIM), lambda i: (i, 0)),
            pl.BlockSpec((2 * DIM, 2 * DIM), lambda i: (0, 0)),
            pl.BlockSpec((1, 2 * DIM), lambda i: (0, 0)),
            pl.BlockSpec((2 * DIM, DIM), lambda i: (0, 0)),
            pl.BlockSpec((1, DIM), lambda i: (0, 0)),
        ],
        out_specs=pl.BlockSpec((1, DIM, B), lambda i: (i, 0, 0)),
        out_shape=jax.ShapeDtypeStruct((L, DIM, B), jnp.float32),
        interpret=interpret,
    )(x2, W1, b1, W2, b2)


def kernel(fine_ids, coarse_ids, fine_table, coarse_table, W1, b1, W2, b2):
    fiT = fine_ids.astype(jnp.int32).T.reshape(NL) * 2
    ciT = coarse_ids.astype(jnp.int32).T.reshape(NL) * 2
    ftab_v = (jnp.zeros((100008, 2 * DIM), jnp.bfloat16)
              .at[:FROWS, :DIM].set(fine_table.astype(jnp.bfloat16))
              .reshape(200016, DIM))
    ctab_v = jnp.pad(coarse_table.astype(jnp.bfloat16),
                     ((0, 7), (0, DIM))).reshape(2016, DIM)
    x2 = _sc_gather(fiT, ciT, ftab_v, ctab_v)
    outT = _tc_mlp(x2, W1.astype(jnp.bfloat16), b1.reshape(1, 2 * DIM),
                   W2, b2.reshape(1, DIM))
    return jnp.transpose(outT, (2, 0, 1)), jnp.float32(0.5)


# speedup vs baseline: 2.1154x; 2.1154x over previous
import jax
import jax.numpy as jnp
from jax import lax
from jax.experimental import pallas as pl
from jax.experimental.pallas import tpu as pltpu
from jax.experimental.pallas import tpu_sc as plsc

B, L, DIM = 4096, 50, 64
NL = B * L
NC, NS = 2, 16
NW = NC * NS
PER_W = NL // NW
TCH = 128
NCH = PER_W // TCH
NBUF = 5
FROWS = 100001


def _sc_gather_body(fidx_hbm, cidx_hbm, ftab_hbm, ctab_hbm, x_hbm,
                    ftmp, ctmp, fbuf, cbuf, fsems, csems):
    wid = lax.axis_index("s") * NC + lax.axis_index("c")
    tbase = wid * PER_W
    pltpu.sync_copy(fidx_hbm.at[pl.ds(tbase, PER_W)], ftmp)
    pltpu.sync_copy(cidx_hbm.at[pl.ds(tbase, PER_W)], ctmp)

    def gather(j, slot):
        jc = lax.min(j, NCH - 1)
        pltpu.async_copy(ftab_hbm.at[ftmp.at[pl.ds(jc * TCH, TCH)]],
                         fbuf.at[slot], fsems.at[slot])
        pltpu.async_copy(ctab_hbm.at[ctmp.at[pl.ds(jc * TCH, TCH)]],
                         cbuf.at[slot], csems.at[slot])

    def wait_write(j, slot):
        pltpu.make_async_copy(ftab_hbm.at[ftmp.at[pl.ds(0, TCH)]],
                              fbuf.at[slot], fsems.at[slot]).wait()
        pltpu.make_async_copy(ctab_hbm.at[ctmp.at[pl.ds(0, TCH)]],
                              cbuf.at[slot], csems.at[slot]).wait()
        row0 = tbase + j * TCH
        pltpu.sync_copy(fbuf.at[slot],
                        x_hbm.at[pl.ds(row0, TCH), pl.ds(0, DIM)])
        pltpu.sync_copy(cbuf.at[slot],
                        x_hbm.at[pl.ds(row0, TCH), pl.ds(DIM, DIM)])

    for s in range(NBUF):
        gather(s, s)

    def group(jj, carry):
        j0 = jj * NBUF
        for s in range(NBUF):
            wait_write(j0 + s, s)
            gather(j0 + s + NBUF, s)
        return carry

    lax.fori_loop(0, NCH // NBUF, group, 0)
    for slot in range(NBUF):
        pltpu.make_async_copy(ftab_hbm.at[ftmp.at[pl.ds(0, TCH)]],
                              fbuf.at[slot], fsems.at[slot]).wait()
        pltpu.make_async_copy(ctab_hbm.at[ctmp.at[pl.ds(0, TCH)]],
                              cbuf.at[slot], csems.at[slot]).wait()


def _sc_gather(fidx, cidx, ftab, ctab):
    return pl.kernel(
        _sc_gather_body,
        out_type=jax.ShapeDtypeStruct((NL, 2 * DIM), jnp.float32),
        mesh=plsc.VectorSubcoreMesh(core_axis_name="c", subcore_axis_name="s",
                                    num_cores=NC, num_subcores=NS),
        scratch_types=[
            pltpu.VMEM((PER_W,), jnp.int32),
            pltpu.VMEM((PER_W,), jnp.int32),
            pltpu.VMEM((NBUF, TCH, DIM), jnp.float32),
            pltpu.VMEM((NBUF, TCH, DIM), jnp.float32),
            pltpu.SemaphoreType.DMA((NBUF,)),
            pltpu.SemaphoreType.DMA((NBUF,)),
        ],
        compiler_params=pltpu.CompilerParams(use_tc_tiling_on_sc=False),
    )(fidx, cidx, ftab, ctab)


def _tc_mlp_body(x_ref, w1_ref, b1_ref, w2_ref, b2_ref, o_ref):
    x = x_ref[...]
    h = jnp.dot(x, w1_ref[...], preferred_element_type=jnp.float32)
    h = jnp.maximum(h + b1_ref[...], 0.0)
    y = (jnp.dot(h, w2_ref[...], preferred_element_type=jnp.float32)
         + b2_ref[...])
    o_ref[0] = jnp.transpose(y)


def _tc_mlp(x2, W1, b1, W2, b2, *, interpret=False):
    return pl.pallas_call(
        _tc_mlp_body,
        grid=(L,),
        in_specs=[
            pl.BlockSpec((B, 2 * DIM), lambda i: (i, 0)),
            pl.BlockSpec((2 * DIM, 2 * DIM), lambda i: (0, 0)),
            pl.BlockSpec((1, 2 * DIM), lambda i: (0, 0)),
            pl.BlockSpec((2 * DIM, DIM), lambda i: (0, 0)),
            pl.BlockSpec((1, DIM), lambda i: (0, 0)),
        ],
        out_specs=pl.BlockSpec((1, DIM, B), lambda i: (i, 0, 0)),
        out_shape=jax.ShapeDtypeStruct((L, DIM, B), jnp.float32),
        interpret=interpret,
    )(x2, W1, b1, W2, b2)


def kernel(fine_ids, coarse_ids, fine_table, coarse_table, W1, b1, W2, b2):
    fiT = fine_ids.astype(jnp.int32).T.reshape(NL) * 2
    ciT = coarse_ids.astype(jnp.int32).T.reshape(NL) * 2
    ftab_v = (jnp.zeros((100008, 2 * DIM), jnp.float32)
              .at[:FROWS, :DIM].set(fine_table).reshape(200016, DIM))
    ctab_v = jnp.pad(coarse_table, ((0, 7), (0, DIM))).reshape(2016, DIM)
    x2 = _sc_gather(fiT, ciT, ftab_v, ctab_v)
    outT = _tc_mlp(x2, W1, b1.reshape(1, 2 * DIM), W2, b2.reshape(1, DIM))
    return jnp.transpose(outT, (2, 0, 1)), jnp.float32(0.5)
